# baseline (speedup 1.0000x reference)
import functools
import jax
import jax.numpy as jnp
from jax import lax
from jax.experimental import pallas as pl
from jax.experimental.pallas import tpu as pltpu

NUM_SP = 256
TH = 8


def _fused_body(xm_ref, x0_ref, xp_ref, sm_ref, s0_ref, sp_ref,
                im_ref, i0_ref, ip_ref, attn_ref,
                Wv_ref, bv_ref, Wp_ref, bp_ref, out_ref, *, H, W, HD, hd, K, NSP):
    C = HD * hd
    P = K // 2
    HALO = TH + 2 * P
    i = pl.program_id(1)

    x_loc = jnp.concatenate(
        [xm_ref[0, TH - P:], x0_ref[0], xp_ref[0, :P]], axis=0)
    sims_loc = jnp.concatenate(
        [sm_ref[0, TH - P:], s0_ref[0], sp_ref[0, :P]], axis=0)
    sinds_loc = jnp.concatenate(
        [im_ref[0, TH - P:], i0_ref[0], ip_ref[0, :P]], axis=0)

    glob = TH * i - P + lax.broadcasted_iota(jnp.int32, (HALO, W, 1), 0)
    row_ok = jnp.logical_and(glob >= 0, glob < H)
    sims_loc = jnp.where(row_ok, sims_loc, 0.0)

    v_loc = jnp.dot(x_loc.reshape(HALO * W, C), Wv_ref[...],
                    preferred_element_type=jnp.float32) + bv_ref[...]
    v_loc = v_loc.reshape(HALO, W, C)

    sp_iota = lax.broadcasted_iota(jnp.int32, (HALO, W, NUM_SP), 2)
    S = jnp.zeros((HALO, W, NUM_SP), jnp.float32)
    for q in range(NSP):
        hit = sinds_loc[:, :, q:q + 1] == sp_iota
        S = S + jnp.where(hit, sims_loc[:, :, q:q + 1], 0.0)
    S_bf = S.astype(jnp.bfloat16)
    S_c = S_bf[P:P + TH]
    S_pad = jnp.concatenate(
        [jnp.zeros((HALO, P, NUM_SP), jnp.bfloat16), S_bf,
         jnp.zeros((HALO, P, NUM_SP), jnp.bfloat16)], axis=1)

    attn_loc = attn_ref[0].reshape(TH * W, K * K * HD)

    e_row = lax.broadcasted_iota(jnp.int32, (HD, C), 0)
    e_col = lax.broadcasted_iota(jnp.int32, (HD, C), 1) // hd
    E = (e_row == e_col).astype(jnp.bfloat16)

    ones_hd = jnp.ones((NUM_SP, HD), jnp.bfloat16)

    acc = jnp.zeros((TH, W, C), jnp.float32)
    for dx in range(K):
        sh = dx - P
        S_sh = S_pad[:, dx:dx + W]
        v_sh = v_loc if sh == 0 else jnp.roll(v_loc, -sh, axis=1)
        for dy in range(K):
            idx = dy * K + dx
            S_n = S_sh[dy:dy + TH]
            v_n = v_sh[dy:dy + TH]
            wgt6 = jnp.dot((S_c * S_n).reshape(TH * W, NUM_SP), ones_hd,
                           preferred_element_type=jnp.float32
                           ).astype(jnp.bfloat16)
            cw = attn_loc[:, idx * HD:(idx + 1) * HD] * wgt6
            u = jnp.dot(cw, E,
                        preferred_element_type=jnp.float32).reshape(TH, W, C)
            acc = acc + u * v_n

    out = jnp.dot(acc.reshape(TH * W, C), Wp_ref[...],
                  preferred_element_type=jnp.float32) + bp_ref[...]
    out_ref[0] = out.reshape(TH, W, C)


def kernel(x, attn, sims, sinds, Wv, bv, Wp, bp):
    B, H, W, C = x.shape
    HD = attn.shape[1]
    hd = C // HD
    K = 7
    NSP = sims.shape[-1]
    assert H % TH == 0
    NT = H // TH

    attn2 = attn.transpose(0, 2, 3, 4, 1).reshape(B, H, W, K * K * HD)
    attn2 = attn2.astype(jnp.bfloat16)
    bv2 = bv.reshape(1, C)
    bp2 = bp.reshape(1, C)

    body = functools.partial(_fused_body, H=H, W=W, HD=HD, hd=hd, K=K, NSP=NSP)

    def hm_m(b, i):
        return (b, jnp.maximum(i - 1, 0), 0, 0)

    def hm(b, i):
        return (b, i, 0, 0)

    def hm_p(b, i):
        return (b, jnp.minimum(i + 1, NT - 1), 0, 0)

    def wspec(lastdim):
        return [pl.BlockSpec((1, TH, W, lastdim), m) for m in (hm_m, hm, hm_p)]

    grid = (B, NT)
    out = pl.pallas_call(
        body,
        grid=grid,
        in_specs=(wspec(C) + wspec(NSP) + wspec(NSP) + [
            pl.BlockSpec((1, TH, W, K * K * HD), hm),
            pl.BlockSpec((C, C), lambda b, i: (0, 0)),
            pl.BlockSpec((1, C), lambda b, i: (0, 0)),
            pl.BlockSpec((C, C), lambda b, i: (0, 0)),
            pl.BlockSpec((1, C), lambda b, i: (0, 0)),
        ]),
        out_specs=pl.BlockSpec((1, TH, W, C), hm),
        out_shape=jax.ShapeDtypeStruct((B, H, W, C), jnp.float32),
        compiler_params=pltpu.CompilerParams(
            dimension_semantics=("parallel", "arbitrary"),
            vmem_limit_bytes=100 * 1024 * 1024,
        ),
    )(x, x, x, sims, sims, sims, sinds, sinds, sinds, attn2, Wv, bv2, Wp, bp2)
    return out


# speedup vs baseline: 1.3926x; 1.3926x over previous
import functools
import jax
import jax.numpy as jnp
from jax import lax
from jax.experimental import pallas as pl
from jax.experimental.pallas import tpu as pltpu

NUM_SP = 256
TH = 8


def _fused_body(xm_ref, x0_ref, xp_ref, sm_ref, s0_ref, sp_ref,
                im_ref, i0_ref, ip_ref, attn_ref,
                Wv_ref, bv_ref, Wp_ref, bp_ref, out_ref, *, H, W, HD, hd, K, NSP):
    C = HD * hd
    P = K // 2
    HALO = TH + 2 * P
    i = pl.program_id(1)

    x_loc = jnp.concatenate(
        [xm_ref[0, TH - P:], x0_ref[0], xp_ref[0, :P]], axis=0)
    sims_loc = jnp.concatenate(
        [sm_ref[0, TH - P:], s0_ref[0], sp_ref[0, :P]], axis=0)
    sinds_loc = jnp.concatenate(
        [im_ref[0, TH - P:], i0_ref[0], ip_ref[0, :P]], axis=0)

    glob = TH * i - P + lax.broadcasted_iota(jnp.int32, (HALO, W, 1), 0)
    row_ok = jnp.logical_and(glob >= 0, glob < H)
    sims_loc = jnp.where(row_ok, sims_loc, 0.0)

    v_loc = jnp.dot(x_loc.reshape(HALO * W, C), Wv_ref[...],
                    preferred_element_type=jnp.float32) + bv_ref[...]
    v_loc = v_loc.reshape(HALO, W, C)

    sp_iota = lax.broadcasted_iota(jnp.int32, (HALO, W, NUM_SP), 2)
    S = jnp.zeros((HALO, W, NUM_SP), jnp.float32)
    for q in range(NSP):
        onehot = (sinds_loc[:, :, q:q + 1] == sp_iota).astype(jnp.float32)
        S = S + sims_loc[:, :, q:q + 1] * onehot
    S_bf = S.astype(jnp.bfloat16)
    S_c = S_bf[P:P + TH]
    S_pad = jnp.concatenate(
        [jnp.zeros((HALO, P, NUM_SP), jnp.bfloat16), S_bf,
         jnp.zeros((HALO, P, NUM_SP), jnp.bfloat16)], axis=1)

    attn_loc = attn_ref[0].reshape(TH * W, K * K * HD)

    e_row = lax.broadcasted_iota(jnp.int32, (HD, C), 0)
    e_col = lax.broadcasted_iota(jnp.int32, (HD, C), 1) // hd
    E = (e_row == e_col).astype(jnp.bfloat16)

    ones_bc = jnp.ones((NUM_SP, C), jnp.bfloat16)

    acc = jnp.zeros((TH, W, C), jnp.float32)
    for dx in range(K):
        sh = dx - P
        S_sh = S_pad[:, dx:dx + W]
        v_sh = v_loc if sh == 0 else jnp.roll(v_loc, -sh, axis=1)
        for dy in range(K):
            idx = dy * K + dx
            S_n = S_sh[dy:dy + TH]
            v_n = v_sh[dy:dy + TH]
            wgt_b = jnp.dot((S_c * S_n).reshape(TH * W, NUM_SP), ones_bc,
                            preferred_element_type=jnp.float32
                            ).reshape(TH, W, C)
            a = jnp.dot(attn_loc[:, idx * HD:(idx + 1) * HD], E,
                        preferred_element_type=jnp.float32).reshape(TH, W, C)
            acc = acc + (a * wgt_b) * v_n

    out = jnp.dot(acc.reshape(TH * W, C), Wp_ref[...],
                  preferred_element_type=jnp.float32) + bp_ref[...]
    out_ref[0] = out.reshape(TH, W, C)


def kernel(x, attn, sims, sinds, Wv, bv, Wp, bp):
    B, H, W, C = x.shape
    HD = attn.shape[1]
    hd = C // HD
    K = 7
    NSP = sims.shape[-1]
    assert H % TH == 0
    NT = H // TH

    attn2 = attn.transpose(0, 2, 3, 4, 1).reshape(B, H, W, K * K * HD)
    attn2 = attn2.astype(jnp.bfloat16)
    bv2 = bv.reshape(1, C)
    bp2 = bp.reshape(1, C)

    body = functools.partial(_fused_body, H=H, W=W, HD=HD, hd=hd, K=K, NSP=NSP)

    def hm_m(b, i):
        return (b, jnp.maximum(i - 1, 0), 0, 0)

    def hm(b, i):
        return (b, i, 0, 0)

    def hm_p(b, i):
        return (b, jnp.minimum(i + 1, NT - 1), 0, 0)

    def wspec(lastdim):
        return [pl.BlockSpec((1, TH, W, lastdim), m) for m in (hm_m, hm, hm_p)]

    grid = (B, NT)
    out = pl.pallas_call(
        body,
        grid=grid,
        in_specs=(wspec(C) + wspec(NSP) + wspec(NSP) + [
            pl.BlockSpec((1, TH, W, K * K * HD), hm),
            pl.BlockSpec((C, C), lambda b, i: (0, 0)),
            pl.BlockSpec((1, C), lambda b, i: (0, 0)),
            pl.BlockSpec((C, C), lambda b, i: (0, 0)),
            pl.BlockSpec((1, C), lambda b, i: (0, 0)),
        ]),
        out_specs=pl.BlockSpec((1, TH, W, C), hm),
        out_shape=jax.ShapeDtypeStruct((B, H, W, C), jnp.float32),
        compiler_params=pltpu.CompilerParams(
            dimension_semantics=("parallel", "arbitrary"),
            vmem_limit_bytes=100 * 1024 * 1024,
        ),
    )(x, x, x, sims, sims, sims, sinds, sinds, sinds, attn2, Wv, bv2, Wp, bp2)
    return out
